# baseline (speedup 1.0000x reference)
import functools

import jax
import jax.numpy as jnp
from jax.experimental import pallas as pl
from jax.experimental.pallas import tpu as pltpu

RATIO_POS = 3
NUM_CLASSES = 21
B, A = 32, 8732
MININT = -2147483648


def _box_kernel(gt_ref, pr_ref, lab_ref, box_ref, acc_s):
    b = pl.program_id(0)

    @pl.when(b == 0)
    def _init():
        acc_s[0] = 0.0
        acc_s[1] = 0.0

    gt = gt_ref[0]
    pr = pr_ref[0]
    lab = lab_ref[0]

    posf = (lab > 0).astype(jnp.float32)
    d = pr - gt
    ad = jnp.abs(d)
    sl1 = jnp.where(ad < 1.0, 0.5 * d * d, ad - 0.5)
    acc_s[0] += jnp.sum(jnp.sum(sl1, axis=0, keepdims=True) * posf)
    acc_s[1] += jnp.sum((jnp.sum(gt, axis=0, keepdims=True) > 0)
                        .astype(jnp.float32))

    @pl.when(b == pl.num_programs(0) - 1)
    def _out():
        box_ref[0] = acc_s[0]
        box_ref[1] = acc_s[1]


def _conf_kernel(lab_ref, logit_ref, cls_ref, conf_s, key_s, acc_s):
    b = pl.program_id(0)

    @pl.when(b == 0)
    def _init():
        acc_s[0] = 0.0

    lab = lab_ref[0]
    x = logit_ref[0]

    pos = lab > 0

    lse = jnp.log(jnp.sum(jnp.exp(x), axis=0, keepdims=True))
    cls_iota = jax.lax.broadcasted_iota(jnp.int32, x.shape, 0)
    chosen = jnp.sum(jnp.where(cls_iota == lab, x, 0.0), axis=0,
                     keepdims=True)
    conf = jnp.maximum(lse - chosen, 0.0)

    acc_s[0] += jnp.sum(conf * pos.astype(jnp.float32))

    aidx = jax.lax.broadcasted_iota(jnp.int32, (1, A), 1)
    confbits = jax.lax.bitcast_convert_type(conf, jnp.int32)
    key = jnp.where(pos, -(aidx + 1), confbits)

    conf_s[pl.ds(b, 1), :] = conf
    key_s[pl.ds(b, 1), :] = key

    @pl.when(b == pl.num_programs(0) - 1)
    def _mine():
        keys = key_s[:, :]
        confs = conf_s[:, :]
        t = RATIO_POS * jnp.sum((keys < 0).astype(jnp.int32), axis=1,
                                keepdims=True)

        def step(i, p):
            bit = jax.lax.shift_left(jnp.int32(1), jnp.int32(31) - i)
            cand = p | bit
            cnt = jnp.sum((keys >= (cand ^ MININT)).astype(jnp.int32),
                          axis=1, keepdims=True)
            return jnp.where(cnt >= t, cand, p)

        p = jax.lax.fori_loop(0, 32, step, jnp.zeros((B, 1), jnp.int32))
        theta = p ^ MININT

        gt_m = keys > theta
        eq_m = keys == theta
        c_gt = jnp.sum(gt_m.astype(jnp.float32), axis=1, keepdims=True)
        c_eq = jnp.sum(eq_m.astype(jnp.float32), axis=1, keepdims=True)
        s_gt = jnp.sum(jnp.where(gt_m, confs, 0.0), axis=1, keepdims=True)
        s_eq = jnp.sum(jnp.where(eq_m, confs, 0.0), axis=1, keepdims=True)
        tie = jnp.where(c_eq > 0.0,
                        (t.astype(jnp.float32) - c_gt) * s_eq
                        / jnp.where(c_eq > 0.0, c_eq, 1.0),
                        0.0)
        s_bg = jnp.sum(s_gt + tie)
        cls_ref[0] = acc_s[0] + s_bg


@functools.partial(jax.jit, static_argnames=("interpret",))
def kernel(gt_bboxes, gt_labels, pred_bboxes, pred_labels, interpret=False):
    gt_t = jnp.transpose(gt_bboxes, (0, 2, 1))
    pr_t = jnp.transpose(pred_bboxes, (0, 2, 1))
    lab3 = gt_labels.reshape(B, 1, A).astype(jnp.int32)
    logit_t = jnp.transpose(pred_labels, (0, 2, 1))

    box = pl.pallas_call(
        _box_kernel,
        grid=(B,),
        in_specs=[
            pl.BlockSpec((1, 4, A), lambda b: (b, 0, 0)),
            pl.BlockSpec((1, 4, A), lambda b: (b, 0, 0)),
            pl.BlockSpec((1, 1, A), lambda b: (b, 0, 0)),
        ],
        out_specs=pl.BlockSpec(memory_space=pltpu.SMEM),
        out_shape=jax.ShapeDtypeStruct((2,), jnp.float32),
        scratch_shapes=[pltpu.SMEM((2,), jnp.float32)],
        interpret=interpret,
    )(gt_t, pr_t, lab3)

    cls = pl.pallas_call(
        _conf_kernel,
        grid=(B,),
        in_specs=[
            pl.BlockSpec((1, 1, A), lambda b: (b, 0, 0)),
            pl.BlockSpec((1, NUM_CLASSES, A), lambda b: (b, 0, 0)),
        ],
        out_specs=pl.BlockSpec(memory_space=pltpu.SMEM),
        out_shape=jax.ShapeDtypeStruct((1,), jnp.float32),
        scratch_shapes=[
            pltpu.VMEM((B, A), jnp.float32),
            pltpu.VMEM((B, A), jnp.int32),
            pltpu.SMEM((1,), jnp.float32),
        ],
        interpret=interpret,
    )(lab3, logit_t)
    return (box[0] / box[1], cls[0] / box[1])


# speedup vs baseline: 1.0105x; 1.0105x over previous
import functools

import jax
import jax.numpy as jnp
from jax.experimental import pallas as pl
from jax.experimental.pallas import tpu as pltpu

RATIO_POS = 3
NUM_CLASSES = 21
B, A = 32, 8732
RPS = 8
SLAB = RPS * NUM_CLASSES
MININT = -2147483648


def _box_kernel(gt_ref, pr_ref, lab_ref, box_ref):
    gt = gt_ref[...]
    pr = pr_ref[...]
    lab = lab_ref[...]

    posf = (lab > 0).astype(jnp.float32)
    j4 = jax.lax.broadcasted_iota(jnp.int32, (4 * B, B), 0)
    i4 = jax.lax.broadcasted_iota(jnp.int32, (4 * B, B), 1)
    e4 = (j4 // 4 == i4).astype(jnp.float32)
    pos4 = jnp.dot(e4, posf, preferred_element_type=jnp.float32)

    d = pr - gt
    ad = jnp.abs(d)
    sl1 = jnp.where(ad < 1.0, 0.5 * d * d, ad - 0.5)
    box_ref[0] = jnp.sum(sl1 * pos4)

    gsum = jnp.dot(e4.T, gt, preferred_element_type=jnp.float32)
    box_ref[1] = jnp.sum((gsum > 0.0).astype(jnp.float32))


def _conf_kernel(lab_ref, logit_ref, box_ref, reg_ref, cls_ref,
                 conf_s, key_s):
    g = pl.program_id(0)

    lab = lab_ref[...]
    x = logit_ref[...]

    pos = lab > 0

    r_i = jax.lax.broadcasted_iota(jnp.int32, (RPS, SLAB), 0)
    s_i = jax.lax.broadcasted_iota(jnp.int32, (RPS, SLAB), 1)
    bd = (s_i // NUM_CLASSES == r_i).astype(jnp.float32)

    e = jnp.exp(x)
    sums = jnp.dot(bd, e, preferred_element_type=jnp.float32)

    lab_e = jnp.dot(bd.T, lab.astype(jnp.float32),
                    preferred_element_type=jnp.float32)
    cls_i = jax.lax.broadcasted_iota(jnp.int32, (SLAB, A), 0) % NUM_CLASSES
    sel = jnp.where(cls_i.astype(jnp.float32) == lab_e, x, 0.0)
    chosen = jnp.dot(bd, sel, preferred_element_type=jnp.float32)

    conf = jnp.maximum(jnp.log(sums) - chosen, 0.0)

    aidx = jax.lax.broadcasted_iota(jnp.int32, (RPS, A), 1)
    confbits = jax.lax.bitcast_convert_type(conf, jnp.int32)
    key = jnp.where(pos, -(aidx + 1), confbits)

    conf_s[pl.ds(g * RPS, RPS), :] = conf
    key_s[pl.ds(g * RPS, RPS), :] = key

    @pl.when(g == pl.num_programs(0) - 1)
    def _mine():
        keys = key_s[:, :]
        confs = conf_s[:, :]
        posm = keys < 0
        t = RATIO_POS * jnp.sum(posm.astype(jnp.int32), axis=1,
                                keepdims=True)
        sum_pos_conf = jnp.sum(jnp.where(posm, confs, 0.0))

        def step(i, p):
            bit = jax.lax.shift_left(jnp.int32(1), jnp.int32(31) - i)
            cand = p | bit
            cnt = jnp.sum((keys >= (cand ^ MININT)).astype(jnp.int32),
                          axis=1, keepdims=True)
            return jnp.where(cnt >= t, cand, p)

        p = jax.lax.fori_loop(0, 32, step, jnp.zeros((B, 1), jnp.int32))
        theta = p ^ MININT

        gt_m = keys > theta
        eq_m = keys == theta
        c_gt = jnp.sum(gt_m.astype(jnp.float32), axis=1, keepdims=True)
        c_eq = jnp.sum(eq_m.astype(jnp.float32), axis=1, keepdims=True)
        s_gt = jnp.sum(jnp.where(gt_m, confs, 0.0), axis=1, keepdims=True)
        s_eq = jnp.sum(jnp.where(eq_m, confs, 0.0), axis=1, keepdims=True)
        tie = jnp.where(c_eq > 0.0,
                        (t.astype(jnp.float32) - c_gt) * s_eq
                        / jnp.where(c_eq > 0.0, c_eq, 1.0),
                        0.0)
        s_bg = jnp.sum(s_gt + tie)

        num_pos = box_ref[1]
        reg_ref[0] = box_ref[0] / num_pos
        cls_ref[0] = (sum_pos_conf + s_bg) / num_pos


@functools.partial(jax.jit, static_argnames=("interpret",))
def kernel(gt_bboxes, gt_labels, pred_bboxes, pred_labels, interpret=False):
    lab32 = gt_labels.astype(jnp.int32)
    gt4 = jnp.transpose(gt_bboxes, (0, 2, 1)).reshape(4 * B, A)
    pr4 = jnp.transpose(pred_bboxes, (0, 2, 1)).reshape(4 * B, A)
    logit_t = jnp.transpose(pred_labels, (0, 2, 1)).reshape(B * NUM_CLASSES, A)

    box = pl.pallas_call(
        _box_kernel,
        in_specs=[
            pl.BlockSpec((4 * B, A), lambda: (0, 0)),
            pl.BlockSpec((4 * B, A), lambda: (0, 0)),
            pl.BlockSpec((B, A), lambda: (0, 0)),
        ],
        out_specs=pl.BlockSpec(memory_space=pltpu.SMEM),
        out_shape=jax.ShapeDtypeStruct((2,), jnp.float32),
        interpret=interpret,
    )(gt4, pr4, lab32)

    reg, cls = pl.pallas_call(
        _conf_kernel,
        grid=(B // RPS,),
        in_specs=[
            pl.BlockSpec((RPS, A), lambda g: (g, 0)),
            pl.BlockSpec((SLAB, A), lambda g: (g, 0)),
            pl.BlockSpec(memory_space=pltpu.SMEM),
        ],
        out_specs=[
            pl.BlockSpec(memory_space=pltpu.SMEM),
            pl.BlockSpec(memory_space=pltpu.SMEM),
        ],
        out_shape=[
            jax.ShapeDtypeStruct((1,), jnp.float32),
            jax.ShapeDtypeStruct((1,), jnp.float32),
        ],
        scratch_shapes=[
            pltpu.VMEM((B, A), jnp.float32),
            pltpu.VMEM((B, A), jnp.int32),
        ],
        interpret=interpret,
    )(lab32, logit_t, box)
    return (reg[0], cls[0])


# speedup vs baseline: 1.0517x; 1.0407x over previous
import functools

import jax
import jax.numpy as jnp
from jax.experimental import pallas as pl
from jax.experimental.pallas import tpu as pltpu

RATIO_POS = 3
NUM_CLASSES = 21
B, A = 32, 8732
RPS = 8
SLAB = RPS * NUM_CLASSES
MININT = -2147483648


def _box_kernel(gt_ref, pr_ref, lab_ref, box_ref):
    gt = gt_ref[...]
    pr = pr_ref[...]
    lab = lab_ref[...]

    posf = (lab > 0).astype(jnp.float32)
    d = pr - gt
    ad = jnp.abs(d)
    sl1 = jnp.where(ad < 1.0, 0.5 * d * d, ad - 0.5)
    box_ref[0] = jnp.sum(jnp.sum(sl1, axis=1) * posf)

    box_ref[1] = jnp.sum((jnp.sum(gt, axis=1) > 0.0).astype(jnp.float32))


def _conf_kernel(lab_ref, logit_ref, box_ref, reg_ref, cls_ref,
                 conf_s, key_s):
    g = pl.program_id(0)

    lab = lab_ref[...]
    x = logit_ref[...]

    pos = lab > 0

    r_i = jax.lax.broadcasted_iota(jnp.int32, (RPS, SLAB), 0)
    s_i = jax.lax.broadcasted_iota(jnp.int32, (RPS, SLAB), 1)
    bd = (s_i // NUM_CLASSES == r_i).astype(jnp.float32)

    e = jnp.exp(x)
    sums = jnp.dot(bd, e, preferred_element_type=jnp.float32)

    lab_e = jnp.dot(bd.T, lab.astype(jnp.float32),
                    preferred_element_type=jnp.float32)
    cls_i = jax.lax.broadcasted_iota(jnp.int32, (SLAB, A), 0) % NUM_CLASSES
    sel = jnp.where(cls_i.astype(jnp.float32) == lab_e, x, 0.0)
    chosen = jnp.dot(bd, sel, preferred_element_type=jnp.float32)

    conf = jnp.maximum(jnp.log(sums) - chosen, 0.0)

    aidx = jax.lax.broadcasted_iota(jnp.int32, (RPS, A), 1)
    confbits = jax.lax.bitcast_convert_type(conf, jnp.int32)
    key = jnp.where(pos, -(aidx + 1), confbits)

    conf_s[pl.ds(g * RPS, RPS), :] = conf
    key_s[pl.ds(g * RPS, RPS), :] = key

    @pl.when(g == pl.num_programs(0) - 1)
    def _mine():
        keys = key_s[:, :]
        confs = conf_s[:, :]
        posm = keys < 0
        t = RATIO_POS * jnp.sum(posm.astype(jnp.int32), axis=1,
                                keepdims=True)
        sum_pos_conf = jnp.sum(jnp.where(posm, confs, 0.0))

        def step(i, p):
            bit = jax.lax.shift_left(jnp.int32(1), jnp.int32(31) - i)
            cand = p | bit
            cnt = jnp.sum((keys >= (cand ^ MININT)).astype(jnp.int32),
                          axis=1, keepdims=True)
            return jnp.where(cnt >= t, cand, p)

        p = jax.lax.fori_loop(0, 32, step, jnp.zeros((B, 1), jnp.int32))
        theta = p ^ MININT

        gt_m = keys > theta
        eq_m = keys == theta
        c_gt = jnp.sum(gt_m.astype(jnp.float32), axis=1, keepdims=True)
        c_eq = jnp.sum(eq_m.astype(jnp.float32), axis=1, keepdims=True)
        s_gt = jnp.sum(jnp.where(gt_m, confs, 0.0), axis=1, keepdims=True)
        s_eq = jnp.sum(jnp.where(eq_m, confs, 0.0), axis=1, keepdims=True)
        tie = jnp.where(c_eq > 0.0,
                        (t.astype(jnp.float32) - c_gt) * s_eq
                        / jnp.where(c_eq > 0.0, c_eq, 1.0),
                        0.0)
        s_bg = jnp.sum(s_gt + tie)

        num_pos = box_ref[1]
        reg_ref[0] = box_ref[0] / num_pos
        cls_ref[0] = (sum_pos_conf + s_bg) / num_pos


@functools.partial(jax.jit, static_argnames=("interpret",))
def kernel(gt_bboxes, gt_labels, pred_bboxes, pred_labels, interpret=False):
    lab32 = gt_labels.astype(jnp.int32)
    gt4 = jnp.transpose(gt_bboxes, (0, 2, 1))
    pr4 = jnp.transpose(pred_bboxes, (0, 2, 1))
    logit_t = jnp.transpose(pred_labels, (0, 2, 1)).reshape(B * NUM_CLASSES, A)

    box = pl.pallas_call(
        _box_kernel,
        in_specs=[
            pl.BlockSpec((B, 4, A), lambda: (0, 0, 0)),
            pl.BlockSpec((B, 4, A), lambda: (0, 0, 0)),
            pl.BlockSpec((B, A), lambda: (0, 0)),
        ],
        out_specs=pl.BlockSpec(memory_space=pltpu.SMEM),
        out_shape=jax.ShapeDtypeStruct((2,), jnp.float32),
        interpret=interpret,
    )(gt4, pr4, lab32)

    reg, cls = pl.pallas_call(
        _conf_kernel,
        grid=(B // RPS,),
        in_specs=[
            pl.BlockSpec((RPS, A), lambda g: (g, 0)),
            pl.BlockSpec((SLAB, A), lambda g: (g, 0)),
            pl.BlockSpec(memory_space=pltpu.SMEM),
        ],
        out_specs=[
            pl.BlockSpec(memory_space=pltpu.SMEM),
            pl.BlockSpec(memory_space=pltpu.SMEM),
        ],
        out_shape=[
            jax.ShapeDtypeStruct((1,), jnp.float32),
            jax.ShapeDtypeStruct((1,), jnp.float32),
        ],
        scratch_shapes=[
            pltpu.VMEM((B, A), jnp.float32),
            pltpu.VMEM((B, A), jnp.int32),
        ],
        interpret=interpret,
    )(lab32, logit_t, box)
    return (reg[0], cls[0])


# speedup vs baseline: 1.0654x; 1.0131x over previous
import functools

import jax
import jax.numpy as jnp
from jax.experimental import pallas as pl
from jax.experimental.pallas import tpu as pltpu

RATIO_POS = 3
NUM_CLASSES = 21
B, A = 32, 8732
MININT = -2147483648


def _box_kernel(gt_ref, pr_ref, lab_ref, box_ref, acc_s):
    b = pl.program_id(0)

    @pl.when(b == 0)
    def _init():
        acc_s[0] = 0.0
        acc_s[1] = 0.0

    gt = gt_ref[0]
    pr = pr_ref[0]
    lab = lab_ref[0]

    posf = (lab > 0).astype(jnp.float32)
    d = pr - gt
    ad = jnp.abs(d)
    sl1 = jnp.where(ad < 1.0, 0.5 * d * d, ad - 0.5)
    acc_s[0] += jnp.sum(jnp.sum(sl1, axis=0, keepdims=True) * posf)
    acc_s[1] += jnp.sum((jnp.sum(gt, axis=0, keepdims=True) > 0)
                        .astype(jnp.float32))

    @pl.when(b == pl.num_programs(0) - 1)
    def _out():
        box_ref[0] = acc_s[0]
        box_ref[1] = acc_s[1]


def _conf_kernel(lab_ref, logit_ref, box_ref, reg_ref, cls_ref,
                 conf_s, key_s, acc_s):
    b = pl.program_id(0)

    @pl.when(b == 0)
    def _init():
        acc_s[0] = 0.0

    lab = lab_ref[0]
    x = logit_ref[0]

    pos = lab > 0

    lse = jnp.log(jnp.sum(jnp.exp(x), axis=0, keepdims=True))
    cls_iota = jax.lax.broadcasted_iota(jnp.int32, x.shape, 0)
    chosen = jnp.sum(jnp.where(cls_iota == lab, x, 0.0), axis=0,
                     keepdims=True)
    conf = jnp.maximum(lse - chosen, 0.0)

    acc_s[0] += jnp.sum(conf * pos.astype(jnp.float32))

    aidx = jax.lax.broadcasted_iota(jnp.int32, (1, A), 1)
    confbits = jax.lax.bitcast_convert_type(conf, jnp.int32)
    key = jnp.where(pos, -(aidx + 1), confbits)

    conf_s[pl.ds(b, 1), :] = conf
    key_s[pl.ds(b, 1), :] = key

    @pl.when(b == pl.num_programs(0) - 1)
    def _mine():
        keys = key_s[:, :]
        confs = conf_s[:, :]
        t = RATIO_POS * jnp.sum((keys < 0).astype(jnp.int32), axis=1,
                                keepdims=True)

        def step(i, p):
            bit = jax.lax.shift_left(jnp.int32(1), jnp.int32(31) - i)
            cand = p | bit
            cnt = jnp.sum((keys >= (cand ^ MININT)).astype(jnp.int32),
                          axis=1, keepdims=True)
            return jnp.where(cnt >= t, cand, p)

        p = jax.lax.fori_loop(0, 32, step, jnp.zeros((B, 1), jnp.int32))
        theta = p ^ MININT

        gt_m = keys > theta
        eq_m = keys == theta
        c_gt = jnp.sum(gt_m.astype(jnp.float32), axis=1, keepdims=True)
        c_eq = jnp.sum(eq_m.astype(jnp.float32), axis=1, keepdims=True)
        s_gt = jnp.sum(jnp.where(gt_m, confs, 0.0), axis=1, keepdims=True)
        s_eq = jnp.sum(jnp.where(eq_m, confs, 0.0), axis=1, keepdims=True)
        tie = jnp.where(c_eq > 0.0,
                        (t.astype(jnp.float32) - c_gt) * s_eq
                        / jnp.where(c_eq > 0.0, c_eq, 1.0),
                        0.0)
        s_bg = jnp.sum(s_gt + tie)

        num_pos = box_ref[1]
        reg_ref[0] = box_ref[0] / num_pos
        cls_ref[0] = (acc_s[0] + s_bg) / num_pos


@functools.partial(jax.jit, static_argnames=("interpret",))
def kernel(gt_bboxes, gt_labels, pred_bboxes, pred_labels, interpret=False):
    gt_t = jnp.transpose(gt_bboxes, (0, 2, 1))
    pr_t = jnp.transpose(pred_bboxes, (0, 2, 1))
    lab3 = gt_labels.reshape(B, 1, A).astype(jnp.int32)
    logit_t = jnp.transpose(pred_labels, (0, 2, 1))

    box = pl.pallas_call(
        _box_kernel,
        grid=(B,),
        in_specs=[
            pl.BlockSpec((1, 4, A), lambda b: (b, 0, 0)),
            pl.BlockSpec((1, 4, A), lambda b: (b, 0, 0)),
            pl.BlockSpec((1, 1, A), lambda b: (b, 0, 0)),
        ],
        out_specs=pl.BlockSpec(memory_space=pltpu.SMEM),
        out_shape=jax.ShapeDtypeStruct((2,), jnp.float32),
        scratch_shapes=[pltpu.SMEM((2,), jnp.float32)],
        interpret=interpret,
    )(gt_t, pr_t, lab3)

    reg, cls = pl.pallas_call(
        _conf_kernel,
        grid=(B,),
        in_specs=[
            pl.BlockSpec((1, 1, A), lambda b: (b, 0, 0)),
            pl.BlockSpec((1, NUM_CLASSES, A), lambda b: (b, 0, 0)),
            pl.BlockSpec(memory_space=pltpu.SMEM),
        ],
        out_specs=[
            pl.BlockSpec(memory_space=pltpu.SMEM),
            pl.BlockSpec(memory_space=pltpu.SMEM),
        ],
        out_shape=[
            jax.ShapeDtypeStruct((1,), jnp.float32),
            jax.ShapeDtypeStruct((1,), jnp.float32),
        ],
        scratch_shapes=[
            pltpu.VMEM((B, A), jnp.float32),
            pltpu.VMEM((B, A), jnp.int32),
            pltpu.SMEM((1,), jnp.float32),
        ],
        interpret=interpret,
    )(lab3, logit_t, box)
    return (reg[0], cls[0])
